# Initial kernel scaffold; baseline (speedup 1.0000x reference)
import jax
import jax.numpy as jnp
from jax.experimental import pallas as pl


def kernel(logits, labels):
    raise NotImplementedError("write your pallas kernel here")


# speedup vs baseline: 39.8179x; 39.8179x over previous
import functools

import jax
import jax.numpy as jnp
from jax import lax
from jax.experimental import pallas as pl
from jax.experimental.pallas import tpu as pltpu
from jax.experimental.pallas import tpu_sc as plsc

B, C, H, W = 4, 21, 512, 512
P = H * W
CP = 24
K = 128
SUB = 42 * K + K
NSUB = 16
CH = NSUB * SUB
NTILES = 32
PCT = 2048
CW = 4096
STRIPE = P // NTILES


def _prep_body(lg_ref, lb_ref, out_ref):
    x = lg_ref[0]
    m = jnp.max(x, axis=0, keepdims=True)
    ex = jnp.exp(x - m)
    p = ex / jnp.sum(ex, axis=0, keepdims=True)
    lab = lb_ref[0]
    cls = lax.broadcasted_iota(jnp.int32, (C, PCT), 0)
    fg = lab == cls
    e = jnp.abs(fg.astype(jnp.float32) - p)
    bin_ = (K - 1) - jnp.clip((e * K).astype(jnp.int32), 0, K - 1)
    row = jnp.where(fg, C, 0) + cls
    lane = lax.broadcasted_iota(jnp.int32, (C, PCT), 1) % NSUB
    idx = lane * SUB + row * K + bin_
    lane24 = lax.broadcasted_iota(jnp.int32, (CP, PCT), 1) % NSUB
    trash = lane24 * SUB + 42 * K
    cls24 = lax.broadcasted_iota(jnp.int32, (CP, PCT), 0)
    idx24 = jnp.concatenate([idx, jnp.zeros((CP - C, PCT), jnp.int32)], axis=0)
    out_ref[0] = jnp.where(cls24 < C, idx24, trash)


def _prep(lg, lb):
    grid = (B, P // PCT)
    return pl.pallas_call(
        _prep_body,
        grid=grid,
        in_specs=[
            pl.BlockSpec((1, C, PCT), lambda b, j: (b, 0, j)),
            pl.BlockSpec((1, 1, PCT), lambda b, j: (b * (P // PCT) + j, 0, 0)),
        ],
        out_specs=pl.BlockSpec((1, CP, PCT), lambda b, j: (b, 0, j)),
        out_shape=jax.ShapeDtypeStruct((B, CP, P), jnp.int32),
    )(lg, lb)


def _sc_hist_body(idx_hbm, out_hbm, buf, hist, sem):
    wid = lax.axis_index("s") * 2 + lax.axis_index("c")
    col0 = wid * STRIPE

    zeros16 = jnp.zeros((16,), jnp.float32)
    ones16 = jnp.ones((16,), jnp.float32)

    def zero_body(i, carry):
        hist[pl.ds(i * 16, 16)] = zeros16
        return carry

    lax.fori_loop(0, CH // 16, zero_body, 0)

    for chunk in range(B * (CP // 8) * (STRIPE // CW)):
        b = chunk // ((CP // 8) * (STRIPE // CW))
        rg = (chunk // (STRIPE // CW)) % (CP // 8)
        cc = chunk % (STRIPE // CW)
        pltpu.sync_copy(
            idx_hbm.at[b, pl.ds(rg * 8, 8), pl.ds(col0 + cc * CW, CW)], buf)

        def grp(i, carry):
            for r in range(8):
                iv = buf[r, pl.ds(i * 16, 16)]
                plsc.addupdate_scatter(hist, [iv], ones16)
            return carry

        lax.fori_loop(0, CW // 16, grp, 0)

    for off in (8, 4, 2, 1):
        def fold_body(i, carry, off=off):
            a = hist[pl.ds(i * 16, 16)]
            bv = hist[pl.ds(i * 16 + off * SUB, 16)]
            hist[pl.ds(i * 16, 16)] = a + bv
            return carry

        lax.fori_loop(0, off * SUB // 16, fold_body, 0)

    pltpu.sync_copy(hist.at[pl.ds(0, SUB)], out_hbm.at[wid])


def _sc_hist(idx):
    mesh = plsc.VectorSubcoreMesh(core_axis_name="c", subcore_axis_name="s")
    f = functools.partial(
        pl.kernel,
        mesh=mesh,
        compiler_params=pltpu.CompilerParams(needs_layout_passes=False),
        out_type=jax.ShapeDtypeStruct((NTILES, SUB), jnp.float32),
        scratch_types=[
            pltpu.VMEM((8, CW), jnp.int32),
            pltpu.VMEM((CH,), jnp.float32),
            pltpu.SemaphoreType.DMA,
        ],
    )(_sc_hist_body)
    return f(idx)


def _loss_body(h_ref, out_ref):
    hs = jnp.sum(h_ref[...], axis=0)
    nbg = hs[:C]
    nfg = hs[C:]
    r = lax.broadcasted_iota(jnp.int32, (K, K), 0)
    c = lax.broadcasted_iota(jnp.int32, (K, K), 1)
    tri = (r <= c).astype(jnp.float32)
    F = jnp.dot(nfg, tri, preferred_element_type=jnp.float32)
    T = jnp.dot(nbg + nfg, tri, preferred_element_type=jnp.float32)
    G = F[:, K - 1:K]
    J = 1.0 - (G - F) / jnp.maximum(G + T - F, 1.0)
    L = jnp.sum(J, axis=1, keepdims=True) / K - J[:, K - 1:K] / (2.0 * K)
    pres = (G > 0).astype(jnp.float32)
    num = jnp.sum(L * pres, axis=0, keepdims=True)
    den = jnp.sum(pres, axis=0, keepdims=True)
    out_ref[...] = num / jnp.maximum(den, 1.0)


def _loss(h):
    return pl.pallas_call(
        _loss_body,
        out_shape=jax.ShapeDtypeStruct((1, 1), jnp.float32),
    )(h)


def kernel(logits, labels):
    lg = logits.reshape(B, C, P)
    lb = labels.reshape(B * (P // PCT), 1, PCT)
    idx = _prep(lg, lb)
    part = _sc_hist(idx)
    out = _loss(part[:, :42 * K].reshape(NTILES, 42, K))
    return out[0, 0]


# speedup vs baseline: 61.5165x; 1.5449x over previous
import functools

import jax
import jax.numpy as jnp
from jax import lax
from jax.experimental import pallas as pl
from jax.experimental.pallas import tpu as pltpu
from jax.experimental.pallas import tpu_sc as plsc

B, C, H, W = 4, 21, 512, 512
P = H * W
CP = 24
K = 128
SUB = 42 * K + K + 15
NSUB = 16
CH = NSUB * SUB + 16
NTILES = 32
PCT = 2048
CW = 2048
STRIPE = P // NTILES


def _prep_body(lg_ref, lb_ref, out_ref):
    x = lg_ref[0]
    m = jnp.max(x, axis=0, keepdims=True)
    ex = jnp.exp(x - m)
    p = ex / jnp.sum(ex, axis=0, keepdims=True)
    lab = lb_ref[0]
    cls = lax.broadcasted_iota(jnp.int32, (C, PCT), 0)
    fg = lab == cls
    e = jnp.abs(fg.astype(jnp.float32) - p)
    bin_ = (K - 1) - jnp.clip((e * K).astype(jnp.int32), 0, K - 1)
    row = jnp.where(fg, C, 0) + cls
    lane = lax.broadcasted_iota(jnp.int32, (C, PCT), 1) % NSUB
    idx = lane * SUB + row * K + bin_
    lane24 = lax.broadcasted_iota(jnp.int32, (CP, PCT), 1) % NSUB
    trash = lane24 * SUB + 42 * K
    cls24 = lax.broadcasted_iota(jnp.int32, (CP, PCT), 0)
    idx24 = jnp.concatenate([idx, jnp.zeros((CP - C, PCT), jnp.int32)], axis=0)
    out_ref[0] = jnp.where(cls24 < C, idx24, trash)


def _prep(lg, lb):
    grid = (B, P // PCT)
    return pl.pallas_call(
        _prep_body,
        grid=grid,
        in_specs=[
            pl.BlockSpec((1, C, PCT), lambda b, j: (b, 0, j)),
            pl.BlockSpec((1, 1, PCT), lambda b, j: (b * (P // PCT) + j, 0, 0)),
        ],
        out_specs=pl.BlockSpec((1, CP, PCT), lambda b, j: (b, 0, j)),
        out_shape=jax.ShapeDtypeStruct((B, CP, P), jnp.int32),
    )(lg, lb)


NCHUNK = B * (CP // 8) * (STRIPE // CW)


def _sc_hist_body(idx_hbm, out_hbm, buf_a, buf_b, hist, sem_a, sem_b):
    wid = lax.axis_index("s") * 2 + lax.axis_index("c")
    col0 = wid * STRIPE

    zeros16 = jnp.zeros((16,), jnp.float32)
    ones16 = jnp.ones((16,), jnp.float32)

    def zero_body(i, carry):
        hist[pl.ds(i * 16, 16)] = zeros16
        return carry

    lax.fori_loop(0, CH // 16, zero_body, 0)

    bufs = (buf_a, buf_b)
    sems = (sem_a, sem_b)

    def src(chunk):
        b = chunk // ((CP // 8) * (STRIPE // CW))
        rg = (chunk // (STRIPE // CW)) % (CP // 8)
        cc = chunk % (STRIPE // CW)
        return idx_hbm.at[b, pl.ds(rg * 8, 8), pl.ds(col0 + cc * CW, CW)]

    pending = pltpu.async_copy(src(0), bufs[0], sems[0])
    for chunk in range(NCHUNK):
        nxt = None
        if chunk + 1 < NCHUNK:
            nxt = pltpu.async_copy(
                src(chunk + 1), bufs[(chunk + 1) % 2], sems[(chunk + 1) % 2])
        pending.wait()
        buf = bufs[chunk % 2]

        def grp(i, carry, buf=buf):
            ivs = [buf[r, pl.ds(i * 16, 16)] for r in range(8)]
            for iv in ivs:
                plsc.addupdate_scatter(hist, [iv], ones16)
            return carry

        lax.fori_loop(0, CW // 16, grp, 0)
        pending = nxt

    nfold = (SUB + 15) // 16
    for blk in range(1, NSUB):
        def fold_body(i, carry, blk=blk):
            a = hist[pl.ds(i * 16, 16)]
            bv = hist[pl.ds(i * 16 + blk * SUB, 16)]
            hist[pl.ds(i * 16, 16)] = a + bv
            return carry

        lax.fori_loop(0, nfold, fold_body, 0)

    pltpu.sync_copy(hist.at[pl.ds(0, 5504)], out_hbm.at[wid])


def _sc_hist(idx):
    mesh = plsc.VectorSubcoreMesh(core_axis_name="c", subcore_axis_name="s")
    f = functools.partial(
        pl.kernel,
        mesh=mesh,
        compiler_params=pltpu.CompilerParams(needs_layout_passes=False),
        out_type=jax.ShapeDtypeStruct((NTILES, 5504), jnp.float32),
        scratch_types=[
            pltpu.VMEM((8, CW), jnp.int32),
            pltpu.VMEM((8, CW), jnp.int32),
            pltpu.VMEM((CH,), jnp.float32),
            pltpu.SemaphoreType.DMA,
            pltpu.SemaphoreType.DMA,
        ],
    )(_sc_hist_body)
    return f(idx)


def _loss_body(h_ref, out_ref):
    hs = jnp.sum(h_ref[...], axis=0)
    nbg = hs[:C]
    nfg = hs[C:]
    r = lax.broadcasted_iota(jnp.int32, (K, K), 0)
    c = lax.broadcasted_iota(jnp.int32, (K, K), 1)
    tri = (r <= c).astype(jnp.float32)
    F = jnp.dot(nfg, tri, preferred_element_type=jnp.float32)
    T = jnp.dot(nbg + nfg, tri, preferred_element_type=jnp.float32)
    G = F[:, K - 1:K]
    J = 1.0 - (G - F) / jnp.maximum(G + T - F, 1.0)
    L = jnp.sum(J, axis=1, keepdims=True) / K - J[:, K - 1:K] / (2.0 * K)
    pres = (G > 0).astype(jnp.float32)
    num = jnp.sum(L * pres, axis=0, keepdims=True)
    den = jnp.sum(pres, axis=0, keepdims=True)
    out_ref[...] = num / jnp.maximum(den, 1.0)


def _loss(h):
    return pl.pallas_call(
        _loss_body,
        out_shape=jax.ShapeDtypeStruct((1, 1), jnp.float32),
    )(h)


def kernel(logits, labels):
    lg = logits.reshape(B, C, P)
    lb = labels.reshape(B * (P // PCT), 1, PCT)
    idx = _prep(lg, lb)
    part = _sc_hist(idx)
    out = _loss(part[:, :42 * K].reshape(NTILES, 42, K))
    return out[0, 0]


# speedup vs baseline: 108.4193x; 1.7624x over previous
import functools

import jax
import jax.numpy as jnp
from jax import lax
from jax.experimental import pallas as pl
from jax.experimental.pallas import tpu as pltpu
from jax.experimental.pallas import tpu_sc as plsc

B, C, H, W = 4, 21, 512, 512
K = 128
SUB = 42 * K + K + 15
NSUB = 16
CH = NSUB * SUB + 16
NTILES = 32
HB = 8
HS = H // NTILES
NCHUNK = B * C


def _prep_body(lg_ref, lb_ref, out_ref):
    x = lg_ref[0]
    ex = jnp.exp(x)
    p = ex / jnp.sum(ex, axis=0, keepdims=True)
    lab = lb_ref[...]
    cls = lax.broadcasted_iota(jnp.int32, (C, HB, W), 0)
    fg = lab == cls
    e = jnp.abs(fg.astype(jnp.float32) - p)
    bin_ = (K - 1) - jnp.clip((e * K).astype(jnp.int32), 0, K - 1)
    row = jnp.where(fg, C, 0) + cls
    lane = lax.broadcasted_iota(jnp.int32, (C, HB, W), 2) % NSUB
    out_ref[0] = lane * SUB + row * K + bin_


def _prep(lg, lb):
    grid = (B, H // HB)
    return pl.pallas_call(
        _prep_body,
        grid=grid,
        in_specs=[
            pl.BlockSpec((1, C, HB, W), lambda b, j: (b, 0, j, 0)),
            pl.BlockSpec((1, HB, W), lambda b, j: (b, j, 0)),
        ],
        out_specs=pl.BlockSpec((1, C, HB, W), lambda b, j: (b, 0, j, 0)),
        out_shape=jax.ShapeDtypeStruct((B, C, H, W), jnp.int32),
    )(lg, lb)


def _sc_hist_body(idx_hbm, out_hbm, buf_a, buf_b, hist, sem_a, sem_b):
    wid = lax.axis_index("s") * 2 + lax.axis_index("c")
    h0 = wid * HS

    zeros16 = jnp.zeros((16,), jnp.float32)
    ones16 = jnp.ones((16,), jnp.float32)

    def zero_body(i, carry):
        hist[pl.ds(i * 16, 16)] = zeros16
        return carry

    lax.fori_loop(0, CH // 16, zero_body, 0)

    bufs = (buf_a, buf_b)
    sems = (sem_a, sem_b)

    def src(chunk):
        b = chunk // C
        c = chunk % C
        return idx_hbm.at[b, c, pl.ds(h0, HS), :]

    pending = pltpu.async_copy(src(0), bufs[0], sems[0])
    for chunk in range(NCHUNK):
        nxt = None
        if chunk + 1 < NCHUNK:
            nxt = pltpu.async_copy(
                src(chunk + 1), bufs[(chunk + 1) % 2], sems[(chunk + 1) % 2])
        pending.wait()
        buf = bufs[chunk % 2]

        def grp(i, carry, buf=buf):
            ivs = [buf[r, pl.ds(i * 16, 16)] for r in range(HS)]
            for iv in ivs:
                plsc.addupdate_scatter(hist, [iv], ones16)
            return carry

        lax.fori_loop(0, W // 16, grp, 0)
        pending = nxt

    nfold = (SUB + 15) // 16
    for blk in range(1, NSUB):
        def fold_body(i, carry, blk=blk):
            a = hist[pl.ds(i * 16, 16)]
            bv = hist[pl.ds(i * 16 + blk * SUB, 16)]
            hist[pl.ds(i * 16, 16)] = a + bv
            return carry

        lax.fori_loop(0, nfold, fold_body, 0)

    pltpu.sync_copy(hist.at[pl.ds(0, 5504)], out_hbm.at[wid])


def _sc_hist(idx):
    mesh = plsc.VectorSubcoreMesh(core_axis_name="c", subcore_axis_name="s")
    f = functools.partial(
        pl.kernel,
        mesh=mesh,
        compiler_params=pltpu.CompilerParams(needs_layout_passes=False),
        out_type=jax.ShapeDtypeStruct((NTILES, 5504), jnp.float32),
        scratch_types=[
            pltpu.VMEM((HS, W), jnp.int32),
            pltpu.VMEM((HS, W), jnp.int32),
            pltpu.VMEM((CH,), jnp.float32),
            pltpu.SemaphoreType.DMA,
            pltpu.SemaphoreType.DMA,
        ],
    )(_sc_hist_body)
    return f(idx)


def _loss_body(h_ref, out_ref):
    hs = jnp.sum(h_ref[...], axis=0)
    nbg = hs[:C]
    nfg = hs[C:]
    r = lax.broadcasted_iota(jnp.int32, (K, K), 0)
    c = lax.broadcasted_iota(jnp.int32, (K, K), 1)
    tri = (r <= c).astype(jnp.float32)
    F = jnp.dot(nfg, tri, preferred_element_type=jnp.float32)
    T = jnp.dot(nbg + nfg, tri, preferred_element_type=jnp.float32)
    G = F[:, K - 1:K]
    J = 1.0 - (G - F) / jnp.maximum(G + T - F, 1.0)
    L = jnp.sum(J, axis=1, keepdims=True) / K - J[:, K - 1:K] / (2.0 * K)
    pres = (G > 0).astype(jnp.float32)
    num = jnp.sum(L * pres, axis=0, keepdims=True)
    den = jnp.sum(pres, axis=0, keepdims=True)
    out_ref[...] = num / jnp.maximum(den, 1.0)


def _loss(h):
    return pl.pallas_call(
        _loss_body,
        out_shape=jax.ShapeDtypeStruct((1, 1), jnp.float32),
    )(h)


def kernel(logits, labels):
    idx = _prep(logits, labels)
    part = _sc_hist(idx)
    out = _loss(part[:, :42 * K].reshape(NTILES, 42, K))
    return out[0, 0]


# speedup vs baseline: 145.5943x; 1.3429x over previous
import functools

import jax
import jax.numpy as jnp
from jax import lax
from jax.experimental import pallas as pl
from jax.experimental.pallas import tpu as pltpu
from jax.experimental.pallas import tpu_sc as plsc

B, C, H, W = 4, 21, 512, 512
K = 128
SUB = 42 * K + K + 15
NSUB = 16
CH = NSUB * SUB + 16
NTILES = 32
HB = 32
HS = H // NTILES
NCHUNK = B * C


def _prep_body(lg_ref, lb_ref, out_ref):
    x = lg_ref[0]
    ex = jnp.exp(x)
    p = ex / jnp.sum(ex, axis=0, keepdims=True)
    lab = lb_ref[...]
    cls = lax.broadcasted_iota(jnp.int32, (C, HB, W), 0)
    fg = lab == cls
    e = jnp.abs(fg.astype(jnp.float32) - p)
    bin_ = (K - 1) - jnp.clip((e * K).astype(jnp.int32), 0, K - 1)
    row = jnp.where(fg, C, 0) + cls
    lane = lax.broadcasted_iota(jnp.int32, (C, HB, W), 2) % NSUB
    out_ref[0] = lane * SUB + row * K + bin_


def _prep(lg, lb):
    grid = (B, H // HB)
    return pl.pallas_call(
        _prep_body,
        grid=grid,
        in_specs=[
            pl.BlockSpec((1, C, HB, W), lambda b, j: (b, 0, j, 0)),
            pl.BlockSpec((1, HB, W), lambda b, j: (b, j, 0)),
        ],
        out_specs=pl.BlockSpec((1, C, HB, W), lambda b, j: (b, 0, j, 0)),
        out_shape=jax.ShapeDtypeStruct((B, C, H, W), jnp.int32),
    )(lg, lb)


def _sc_hist_body(idx_hbm, out_hbm, buf_a, buf_b, hist, sem_a, sem_b):
    wid = lax.axis_index("s") * 2 + lax.axis_index("c")
    h0 = wid * HS

    zeros16 = jnp.zeros((16,), jnp.float32)
    ones16 = jnp.ones((16,), jnp.float32)

    def zero_body(i, carry):
        hist[pl.ds(i * 16, 16)] = zeros16
        return carry

    lax.fori_loop(0, CH // 16, zero_body, 0)

    bufs = (buf_a, buf_b)
    sems = (sem_a, sem_b)

    def src(chunk):
        b = chunk // C
        c = chunk % C
        return idx_hbm.at[b, c, pl.ds(h0, HS), :]

    pending = pltpu.async_copy(src(0), bufs[0], sems[0])
    for chunk in range(NCHUNK):
        nxt = None
        if chunk + 1 < NCHUNK:
            nxt = pltpu.async_copy(
                src(chunk + 1), bufs[(chunk + 1) % 2], sems[(chunk + 1) % 2])
        pending.wait()
        buf = bufs[chunk % 2]

        def grp(i, carry, buf=buf):
            ivs = [buf[r, pl.ds(i * 16, 16)] for r in range(HS)]
            for iv in ivs:
                plsc.addupdate_scatter(hist, [iv], ones16)
            return carry

        lax.fori_loop(0, W // 16, grp, 0)
        pending = nxt

    nfold = (SUB + 15) // 16
    for blk in range(1, NSUB):
        def fold_body(i, carry, blk=blk):
            a = hist[pl.ds(i * 16, 16)]
            bv = hist[pl.ds(i * 16 + blk * SUB, 16)]
            hist[pl.ds(i * 16, 16)] = a + bv
            return carry

        lax.fori_loop(0, nfold, fold_body, 0)

    pltpu.sync_copy(hist.at[pl.ds(0, 5504)], out_hbm.at[wid])


def _sc_hist(idx):
    mesh = plsc.VectorSubcoreMesh(core_axis_name="c", subcore_axis_name="s")
    f = functools.partial(
        pl.kernel,
        mesh=mesh,
        compiler_params=pltpu.CompilerParams(needs_layout_passes=False),
        out_type=jax.ShapeDtypeStruct((NTILES, 5504), jnp.float32),
        scratch_types=[
            pltpu.VMEM((HS, W), jnp.int32),
            pltpu.VMEM((HS, W), jnp.int32),
            pltpu.VMEM((CH,), jnp.float32),
            pltpu.SemaphoreType.DMA,
            pltpu.SemaphoreType.DMA,
        ],
    )(_sc_hist_body)
    return f(idx)


def _loss_body(h_ref, out_ref):
    hs = jnp.sum(h_ref[...], axis=0)
    nbg = hs[:C]
    nfg = hs[C:]
    r = lax.broadcasted_iota(jnp.int32, (K, K), 0)
    c = lax.broadcasted_iota(jnp.int32, (K, K), 1)
    tri = (r <= c).astype(jnp.float32)
    F = jnp.dot(nfg, tri, preferred_element_type=jnp.float32)
    T = jnp.dot(nbg + nfg, tri, preferred_element_type=jnp.float32)
    G = F[:, K - 1:K]
    J = 1.0 - (G - F) / jnp.maximum(G + T - F, 1.0)
    L = jnp.sum(J, axis=1, keepdims=True) / K - J[:, K - 1:K] / (2.0 * K)
    pres = (G > 0).astype(jnp.float32)
    num = jnp.sum(L * pres, axis=0, keepdims=True)
    den = jnp.sum(pres, axis=0, keepdims=True)
    out_ref[...] = num / jnp.maximum(den, 1.0)


def _loss(h):
    return pl.pallas_call(
        _loss_body,
        out_shape=jax.ShapeDtypeStruct((1, 1), jnp.float32),
    )(h)


def kernel(logits, labels):
    idx = _prep(logits, labels)
    part = _sc_hist(idx)
    out = _loss(part[:, :42 * K].reshape(NTILES, 42, K))
    return out[0, 0]


# speedup vs baseline: 156.9282x; 1.0778x over previous
import functools

import jax
import jax.numpy as jnp
from jax import lax
from jax.experimental import pallas as pl
from jax.experimental.pallas import tpu as pltpu
from jax.experimental.pallas import tpu_sc as plsc

B, C, H, W = 4, 21, 512, 512
K = 128
SUB = 42 * K + K + 15
NSUB = 16
CH = NSUB * SUB + 16
NTILES = 32
HB = 64
HS = H // NTILES
NCHUNK = B * C


def _prep_body(lg_ref, lb_ref, out_ref):
    x = lg_ref[0]
    ex = jnp.exp(x)
    p = ex / jnp.sum(ex, axis=0, keepdims=True)
    lab = lb_ref[...]
    cls = lax.broadcasted_iota(jnp.int32, (C, HB, W), 0)
    fg = lab == cls
    e = jnp.abs(fg.astype(jnp.float32) - p)
    bin_ = (K - 1) - jnp.clip((e * K).astype(jnp.int32), 0, K - 1)
    row = jnp.where(fg, C, 0) + cls
    lane = lax.broadcasted_iota(jnp.int32, (C, HB, W), 2) % NSUB
    out_ref[0] = lane * SUB + row * K + bin_


def _prep(lg, lb):
    grid = (B, H // HB)
    return pl.pallas_call(
        _prep_body,
        grid=grid,
        in_specs=[
            pl.BlockSpec((1, C, HB, W), lambda b, j: (b, 0, j, 0)),
            pl.BlockSpec((1, HB, W), lambda b, j: (b, j, 0)),
        ],
        out_specs=pl.BlockSpec((1, C, HB, W), lambda b, j: (b, 0, j, 0)),
        out_shape=jax.ShapeDtypeStruct((B, C, H, W), jnp.int32),
    )(lg, lb)


def _sc_hist_body(idx_hbm, out_hbm, buf_a, buf_b, hist, sem_a, sem_b):
    wid = lax.axis_index("s") * 2 + lax.axis_index("c")
    h0 = wid * HS

    zeros16 = jnp.zeros((16,), jnp.float32)
    ones16 = jnp.ones((16,), jnp.float32)

    def zero_body(i, carry):
        hist[pl.ds(i * 16, 16)] = zeros16
        return carry

    lax.fori_loop(0, CH // 16, zero_body, 0)

    bufs = (buf_a, buf_b)
    sems = (sem_a, sem_b)

    def src(chunk):
        b = chunk // C
        c = chunk % C
        return idx_hbm.at[b, c, pl.ds(h0, HS), :]

    pending = pltpu.async_copy(src(0), bufs[0], sems[0])
    for chunk in range(NCHUNK):
        nxt = None
        if chunk + 1 < NCHUNK:
            nxt = pltpu.async_copy(
                src(chunk + 1), bufs[(chunk + 1) % 2], sems[(chunk + 1) % 2])
        pending.wait()
        buf = bufs[chunk % 2]

        def grp(i, carry, buf=buf):
            ivs = [buf[r, pl.ds(i * 16, 16)] for r in range(HS)]
            for iv in ivs:
                plsc.addupdate_scatter(hist, [iv], ones16)
            return carry

        lax.fori_loop(0, W // 16, grp, 0)
        pending = nxt

    nfold = (SUB + 15) // 16
    for blk in range(1, NSUB):
        def fold_body(i, carry, blk=blk):
            a = hist[pl.ds(i * 16, 16)]
            bv = hist[pl.ds(i * 16 + blk * SUB, 16)]
            hist[pl.ds(i * 16, 16)] = a + bv
            return carry

        lax.fori_loop(0, nfold, fold_body, 0)

    pltpu.sync_copy(hist.at[pl.ds(0, 5504)], out_hbm.at[wid])


def _sc_hist(idx):
    mesh = plsc.VectorSubcoreMesh(core_axis_name="c", subcore_axis_name="s")
    f = functools.partial(
        pl.kernel,
        mesh=mesh,
        compiler_params=pltpu.CompilerParams(needs_layout_passes=False),
        out_type=jax.ShapeDtypeStruct((NTILES, 5504), jnp.float32),
        scratch_types=[
            pltpu.VMEM((HS, W), jnp.int32),
            pltpu.VMEM((HS, W), jnp.int32),
            pltpu.VMEM((CH,), jnp.float32),
            pltpu.SemaphoreType.DMA,
            pltpu.SemaphoreType.DMA,
        ],
    )(_sc_hist_body)
    return f(idx)


def _loss_body(h_ref, out_ref):
    hs = jnp.sum(h_ref[...], axis=0)
    nbg = hs[:C]
    nfg = hs[C:]
    r = lax.broadcasted_iota(jnp.int32, (K, K), 0)
    c = lax.broadcasted_iota(jnp.int32, (K, K), 1)
    tri = (r <= c).astype(jnp.float32)
    F = jnp.dot(nfg, tri, preferred_element_type=jnp.float32)
    T = jnp.dot(nbg + nfg, tri, preferred_element_type=jnp.float32)
    G = F[:, K - 1:K]
    J = 1.0 - (G - F) / jnp.maximum(G + T - F, 1.0)
    L = jnp.sum(J, axis=1, keepdims=True) / K - J[:, K - 1:K] / (2.0 * K)
    pres = (G > 0).astype(jnp.float32)
    num = jnp.sum(L * pres, axis=0, keepdims=True)
    den = jnp.sum(pres, axis=0, keepdims=True)
    out_ref[...] = num / jnp.maximum(den, 1.0)


def _loss(h):
    return pl.pallas_call(
        _loss_body,
        out_shape=jax.ShapeDtypeStruct((1, 1), jnp.float32),
    )(h)


def kernel(logits, labels):
    idx = _prep(logits, labels)
    part = _sc_hist(idx)
    out = _loss(part[:, :42 * K].reshape(NTILES, 42, K))
    return out[0, 0]


# speedup vs baseline: 162.0951x; 1.0329x over previous
import functools

import jax
import jax.numpy as jnp
from jax import lax
from jax.experimental import pallas as pl
from jax.experimental.pallas import tpu as pltpu
from jax.experimental.pallas import tpu_sc as plsc

B, C, H, W = 4, 21, 512, 512
K = 128
SUB = 42 * K + K + 15
NSUB = 16
CH = NSUB * SUB + 16
NTILES = 32
HB = 128
HS = H // NTILES
NCHUNK = B * C


def _prep_body(lg_ref, lb_ref, out_ref):
    x = lg_ref[0]
    ex = jnp.exp(x)
    p = ex / jnp.sum(ex, axis=0, keepdims=True)
    lab = lb_ref[...]
    cls = lax.broadcasted_iota(jnp.int32, (C, HB, W), 0)
    fg = lab == cls
    e = jnp.abs(fg.astype(jnp.float32) - p)
    bin_ = (K - 1) - jnp.clip((e * K).astype(jnp.int32), 0, K - 1)
    row = jnp.where(fg, C, 0) + cls
    lane = lax.broadcasted_iota(jnp.int32, (C, HB, W), 2) % NSUB
    out_ref[0] = lane * SUB + row * K + bin_


def _prep(lg, lb):
    grid = (B, H // HB)
    return pl.pallas_call(
        _prep_body,
        grid=grid,
        in_specs=[
            pl.BlockSpec((1, C, HB, W), lambda b, j: (b, 0, j, 0)),
            pl.BlockSpec((1, HB, W), lambda b, j: (b, j, 0)),
        ],
        out_specs=pl.BlockSpec((1, C, HB, W), lambda b, j: (b, 0, j, 0)),
        out_shape=jax.ShapeDtypeStruct((B, C, H, W), jnp.int32),
    )(lg, lb)


def _sc_hist_body(idx_hbm, out_hbm, buf_a, buf_b, hist, sem_a, sem_b):
    wid = lax.axis_index("s") * 2 + lax.axis_index("c")
    h0 = wid * HS

    zeros16 = jnp.zeros((16,), jnp.float32)
    ones16 = jnp.ones((16,), jnp.float32)

    def zero_body(i, carry):
        hist[pl.ds(i * 16, 16)] = zeros16
        return carry

    lax.fori_loop(0, CH // 16, zero_body, 0)

    bufs = (buf_a, buf_b)
    sems = (sem_a, sem_b)

    def src(chunk):
        b = chunk // C
        c = chunk % C
        return idx_hbm.at[b, c, pl.ds(h0, HS), :]

    pending = pltpu.async_copy(src(0), bufs[0], sems[0])
    for chunk in range(NCHUNK):
        nxt = None
        if chunk + 1 < NCHUNK:
            nxt = pltpu.async_copy(
                src(chunk + 1), bufs[(chunk + 1) % 2], sems[(chunk + 1) % 2])
        pending.wait()
        buf = bufs[chunk % 2]

        def grp(i, carry, buf=buf):
            ivs = [buf[r, pl.ds(i * 16, 16)] for r in range(HS)]
            for iv in ivs:
                plsc.addupdate_scatter(hist, [iv], ones16)
            return carry

        lax.fori_loop(0, W // 16, grp, 0)
        pending = nxt

    nfold = (SUB + 15) // 16
    for blk in range(1, NSUB):
        def fold_body(i, carry, blk=blk):
            a = hist[pl.ds(i * 16, 16)]
            bv = hist[pl.ds(i * 16 + blk * SUB, 16)]
            hist[pl.ds(i * 16, 16)] = a + bv
            return carry

        lax.fori_loop(0, nfold, fold_body, 0)

    pltpu.sync_copy(hist.at[pl.ds(0, 5504)], out_hbm.at[wid])


def _sc_hist(idx):
    mesh = plsc.VectorSubcoreMesh(core_axis_name="c", subcore_axis_name="s")
    f = functools.partial(
        pl.kernel,
        mesh=mesh,
        compiler_params=pltpu.CompilerParams(needs_layout_passes=False),
        out_type=jax.ShapeDtypeStruct((NTILES, 5504), jnp.float32),
        scratch_types=[
            pltpu.VMEM((HS, W), jnp.int32),
            pltpu.VMEM((HS, W), jnp.int32),
            pltpu.VMEM((CH,), jnp.float32),
            pltpu.SemaphoreType.DMA,
            pltpu.SemaphoreType.DMA,
        ],
    )(_sc_hist_body)
    return f(idx)


def _loss_body(h_ref, out_ref):
    hs = jnp.sum(h_ref[...], axis=0)
    nbg = hs[:C]
    nfg = hs[C:]
    r = lax.broadcasted_iota(jnp.int32, (K, K), 0)
    c = lax.broadcasted_iota(jnp.int32, (K, K), 1)
    tri = (r <= c).astype(jnp.float32)
    F = jnp.dot(nfg, tri, preferred_element_type=jnp.float32)
    T = jnp.dot(nbg + nfg, tri, preferred_element_type=jnp.float32)
    G = F[:, K - 1:K]
    J = 1.0 - (G - F) / jnp.maximum(G + T - F, 1.0)
    L = jnp.sum(J, axis=1, keepdims=True) / K - J[:, K - 1:K] / (2.0 * K)
    pres = (G > 0).astype(jnp.float32)
    num = jnp.sum(L * pres, axis=0, keepdims=True)
    den = jnp.sum(pres, axis=0, keepdims=True)
    out_ref[...] = num / jnp.maximum(den, 1.0)


def _loss(h):
    return pl.pallas_call(
        _loss_body,
        out_shape=jax.ShapeDtypeStruct((1, 1), jnp.float32),
    )(h)


def kernel(logits, labels):
    idx = _prep(logits, labels)
    part = _sc_hist(idx)
    out = _loss(part[:, :42 * K].reshape(NTILES, 42, K))
    return out[0, 0]
